# baseline (speedup 1.0000x reference)
import functools
import jax
import jax.numpy as jnp
from jax import lax
from jax.experimental import pallas as pl
from jax.experimental.pallas import tpu as pltpu
from jax.experimental.pallas import tpu_sc as plsc

_G = 4


def _make_sc_kernel(NR, W, H, WC, NBUF):
    info = plsc.get_sparse_core_info()
    NC, NS, L = info.num_cores, info.num_subcores, info.num_lanes
    NW = NC * NS
    rows_per_w = NR // NW
    groups_per_w = rows_per_w // _G
    n_chunks = W // WC
    n_h = H // L
    T = groups_per_w * n_chunks
    assert T % NBUF == 0
    mesh = plsc.VectorSubcoreMesh(core_axis_name="c", subcore_axis_name="s")
    buf_t = pltpu.VMEM((_G, WC, H), jnp.float32)
    scratch = [buf_t] * (2 * NBUF) + [pltpu.SemaphoreType.DMA] * (2 * NBUF)

    @functools.partial(
        pl.kernel,
        mesh=mesh,
        out_type=jax.ShapeDtypeStruct((NR, W, H), jnp.float32),
        scratch_types=scratch,
    )
    def k(x_hbm, o_hbm, *bufsem):
        ibs = bufsem[:NBUF]
        obs = bufsem[NBUF:2 * NBUF]
        sis = bufsem[2 * NBUF:3 * NBUF]
        sos = bufsem[3 * NBUF:]
        wid = lax.axis_index("s") * NC + lax.axis_index("c")
        row0 = wid * rows_per_w

        def task_slc(t):
            g = t // n_chunks
            ci = lax.rem(t, n_chunks)
            return row0 + g * _G, ci * WC

        def start_in(t, ib, sem):
            r, w0 = task_slc(t)
            pltpu.make_async_copy(
                x_hbm.at[pl.ds(r, _G), pl.ds(w0, WC), :], ib, sem).start()

        def wait_in(ib, sem):
            pltpu.make_async_copy(
                x_hbm.at[pl.ds(0, _G), pl.ds(0, WC), :], ib, sem).wait()

        def start_out(t, ob, sem):
            r, w0 = task_slc(t)
            pltpu.make_async_copy(
                ob, o_hbm.at[pl.ds(r, _G), pl.ds(w0, WC), :], sem).start()

        def wait_out(ob, sem):
            pltpu.make_async_copy(
                ob, o_hbm.at[pl.ds(0, _G), pl.ds(0, WC), :], sem).wait()

        def compute(ib, ob):
            def srow(s, c2):
                for kk in range(n_h):
                    off = kk * L
                    v0 = ib[0, s, pl.ds(off, L)]
                    v1 = ib[1, s, pl.ds(off, L)]
                    v2 = ib[2, s, pl.ds(off, L)]
                    v3 = ib[3, s, pl.ds(off, L)]
                    m = jnp.maximum(jnp.maximum(v0, v1), jnp.maximum(v2, v3))
                    z = jnp.zeros((L,), jnp.float32)
                    ob[0, s, pl.ds(off, L)] = jnp.where(v0 == m, v0, z)
                    ob[1, s, pl.ds(off, L)] = jnp.where(v1 == m, v1, z)
                    ob[2, s, pl.ds(off, L)] = jnp.where(v2 == m, v2, z)
                    ob[3, s, pl.ds(off, L)] = jnp.where(v3 == m, v3, z)
                return c2

            lax.fori_loop(0, WC, srow, 0)

        for j in range(NBUF):
            start_in(j, ibs[j], sis[j])

        def rnd(p, carry):
            t = p * NBUF
            for j in range(NBUF):
                tt = t + j
                wait_in(ibs[j], sis[j])

                @pl.when(tt >= NBUF)
                def _():
                    wait_out(obs[j], sos[j])

                compute(ibs[j], obs[j])
                start_out(tt, obs[j], sos[j])

                @pl.when(tt + NBUF < T)
                def _():
                    start_in(tt + NBUF, ibs[j], sis[j])

            return carry

        lax.fori_loop(0, T // NBUF, rnd, 0)
        for j in range(NBUF):
            wait_out(obs[j], sos[j])

    return k


def kernel(x):
    B, C, W, H = x.shape
    NR = B * C
    x3 = x.reshape(NR, W, H)
    out = _make_sc_kernel(NR, W, H, 16, 4)(x3)
    return out.reshape(B, C, W, H)


# speedup vs baseline: 1.0078x; 1.0078x over previous
import functools
import jax
import jax.numpy as jnp
from jax import lax
from jax.experimental import pallas as pl
from jax.experimental.pallas import tpu as pltpu
from jax.experimental.pallas import tpu_sc as plsc

_G = 4


def _make_sc_kernel(NR, W, H, WC, NBUF):
    info = plsc.get_sparse_core_info()
    NC, NS, L = info.num_cores, info.num_subcores, info.num_lanes
    NW = NC * NS
    rows_per_w = NR // NW
    groups_per_w = rows_per_w // _G
    n_chunks = W // WC
    n_h = H // L
    T = groups_per_w * n_chunks
    assert T % NBUF == 0
    mesh = plsc.VectorSubcoreMesh(core_axis_name="c", subcore_axis_name="s")
    buf_t = pltpu.VMEM((_G, WC, H), jnp.float32)
    scratch = [buf_t] * (2 * NBUF) + [pltpu.SemaphoreType.DMA] * (2 * NBUF)

    @functools.partial(
        pl.kernel,
        mesh=mesh,
        out_type=jax.ShapeDtypeStruct((NR, W, H), jnp.float32),
        scratch_types=scratch,
    )
    def k(x_hbm, o_hbm, *bufsem):
        ibs = bufsem[:NBUF]
        obs = bufsem[NBUF:2 * NBUF]
        sis = bufsem[2 * NBUF:3 * NBUF]
        sos = bufsem[3 * NBUF:]
        wid = lax.axis_index("s") * NC + lax.axis_index("c")
        row0 = wid * rows_per_w

        def task_slc(t):
            g = t // n_chunks
            ci = lax.rem(t, n_chunks)
            return row0 + g * _G, ci * WC

        def start_in(t, ib, sem):
            r, w0 = task_slc(t)
            pltpu.make_async_copy(
                x_hbm.at[pl.ds(r, _G), pl.ds(w0, WC), :], ib, sem).start()

        def wait_in(ib, sem):
            pltpu.make_async_copy(
                x_hbm.at[pl.ds(0, _G), pl.ds(0, WC), :], ib, sem).wait()

        def start_out(t, ob, sem):
            r, w0 = task_slc(t)
            pltpu.make_async_copy(
                ob, o_hbm.at[pl.ds(r, _G), pl.ds(w0, WC), :], sem).start()

        def wait_out(ob, sem):
            pltpu.make_async_copy(
                ob, o_hbm.at[pl.ds(0, _G), pl.ds(0, WC), :], sem).wait()

        def compute(ib, ob):
            def srow(s, c2):
                for kk in range(n_h):
                    off = kk * L
                    v0 = ib[0, s, pl.ds(off, L)]
                    v1 = ib[1, s, pl.ds(off, L)]
                    v2 = ib[2, s, pl.ds(off, L)]
                    v3 = ib[3, s, pl.ds(off, L)]
                    m = jnp.maximum(jnp.maximum(v0, v1), jnp.maximum(v2, v3))
                    z = jnp.zeros((L,), jnp.float32)
                    ob[0, s, pl.ds(off, L)] = jnp.where(v0 == m, v0, z)
                    ob[1, s, pl.ds(off, L)] = jnp.where(v1 == m, v1, z)
                    ob[2, s, pl.ds(off, L)] = jnp.where(v2 == m, v2, z)
                    ob[3, s, pl.ds(off, L)] = jnp.where(v3 == m, v3, z)
                return c2

            lax.fori_loop(0, WC, srow, 0)

        for j in range(NBUF):
            start_in(j, ibs[j], sis[j])

        def rnd(p, carry):
            t = p * NBUF
            for j in range(NBUF):
                tt = t + j
                wait_in(ibs[j], sis[j])

                @pl.when(tt >= NBUF)
                def _():
                    wait_out(obs[j], sos[j])

                compute(ibs[j], obs[j])
                start_out(tt, obs[j], sos[j])

                @pl.when(tt + NBUF < T)
                def _():
                    start_in(tt + NBUF, ibs[j], sis[j])

            return carry

        lax.fori_loop(0, T // NBUF, rnd, 0)
        for j in range(NBUF):
            wait_out(obs[j], sos[j])

    return k


def kernel(x):
    B, C, W, H = x.shape
    NR = B * C
    x3 = x.reshape(NR, W, H)
    out = _make_sc_kernel(NR, W, H, 32, 2)(x3)
    return out.reshape(B, C, W, H)
